# baseline (speedup 1.0000x reference)
import jax
import jax.numpy as jnp
from jax.experimental import pallas as pl
from jax.experimental.pallas import tpu as pltpu
from jax.experimental.pallas import tpu_sc as plsc

_NH = 16
_EPS = 1e-10
_L = 128
_GW = 128


def _tree(op, xs):
    xs = list(xs)
    while len(xs) > 1:
        nxt = [op(xs[i], xs[i + 1]) for i in range(0, len(xs) - 1, 2)]
        if len(xs) % 2:
            nxt.append(xs[-1])
        xs = nxt
    return xs[0]


def _select_body(lon_ref, lat_ref, dist_out, idx_out, pidx_out, xidx_out,
                 sq_ref):
    lon = lon_ref[...]
    lat = lat_ref[...]
    r, s = lon.shape
    nc = s // _L
    sq_ref[...] = lon * lon + lat * lat
    lane_iota = jax.lax.broadcasted_iota(jnp.int32, (r, _L), 1)
    big_i = jnp.int32(2 ** 30)

    def chunk(c):
        return sq_ref[:, c * _L:(c + 1) * _L]

    def lane_min():
        return _tree(jnp.minimum, [chunk(c) for c in range(nc)])

    def merge(bv, bi, cv, ci):
        vv = jnp.concatenate([bv, cv], axis=1)
        ii = jnp.concatenate([bi, ci], axis=1)
        nbv, nbi = [], []
        for _ in range(_NH):
            m = jnp.min(vv, axis=1, keepdims=True)
            am = jnp.min(jnp.where(vv == m, ii, big_i), axis=1, keepdims=True)
            nbv.append(m)
            nbi.append(am)
            vv = jnp.where(ii == am, jnp.inf, vv)
        return jnp.concatenate(nbv, 1), jnp.concatenate(nbi, 1)

    def body(carry):
        _, lm, bv, bi = carry
        chunks = [chunk(c) for c in range(nc)]
        pos = _tree(jnp.minimum,
                    [jnp.where(chunks[c] == lm, c, big_i) for c in range(nc)])
        new_chunks = [jnp.where(pos == c, jnp.inf, chunks[c]) for c in range(nc)]
        sq_ref[...] = jnp.concatenate(new_chunks, axis=1)
        bv, bi = merge(bv, bi, lm, pos * _L + lane_iota)
        lm = _tree(jnp.minimum, new_chunks)
        gmin = jnp.min(lm, axis=1)
        done = jnp.all(bv[:, _NH - 1] < gmin)
        return done, lm, bv, bi

    init = (jnp.bool_(False), lane_min(),
            jnp.full((r, _NH), jnp.inf, jnp.float32),
            jnp.zeros((r, _NH), jnp.int32))
    _, _, bv, bi = jax.lax.while_loop(lambda c: jnp.logical_not(c[0]),
                                      body, init)
    dist_out[...] = jnp.sqrt(bv + 1e-12)
    idx_out[...] = bi
    row_ids = pl.program_id(0) * r + jax.lax.broadcasted_iota(jnp.int32, (r, _NH), 0)
    pidx_out[...] = row_ids * (s // _L) + bi // _L
    xidx_out[...] = bi // 8


def _select(d_lon, d_lat):
    t, s = d_lon.shape
    r = min(32, t)
    in_spec = pl.BlockSpec((r, s), lambda i: (i, 0))
    out_spec = pl.BlockSpec((r, _NH), lambda i: (i, 0))
    return pl.pallas_call(
        _select_body,
        grid=(t // r,),
        in_specs=[in_spec, in_spec],
        out_specs=[out_spec] * 4,
        out_shape=(jax.ShapeDtypeStruct((t, _NH), jnp.float32),
                   jax.ShapeDtypeStruct((t, _NH), jnp.int32),
                   jax.ShapeDtypeStruct((t, _NH), jnp.int32),
                   jax.ShapeDtypeStruct((t, _NH), jnp.int32)),
        scratch_shapes=[pltpu.VMEM((r, s), jnp.float32)],
    )(d_lon, d_lat)


def _sc_gather(xt8, lonp, latp, xidx, pidx):
    n = xidx.shape[1]
    mesh = plsc.VectorSubcoreMesh(core_axis_name="core",
                                  subcore_axis_name="subcore")

    @pl.kernel(out_type=(jax.ShapeDtypeStruct((n, 128), jnp.float32),
                         jax.ShapeDtypeStruct((n, 128), jnp.float32),
                         jax.ShapeDtypeStruct((n, 128), jnp.float32)),
               mesh=mesh)
    def gather_kernel(xt_hbm, lonp_hbm, latp_hbm, xi_hbm, pi_hbm,
                      xg_hbm, lg_hbm, tg_hbm):
        def body(xi_vmem, pi_vmem, xg_vmem, lg_vmem, tg_vmem):
            pltpu.sync_copy(xt_hbm.at[xi_vmem.at[0]], xg_vmem)
            pltpu.sync_copy(lonp_hbm.at[pi_vmem.at[0]], lg_vmem)
            pltpu.sync_copy(latp_hbm.at[pi_vmem.at[0]], tg_vmem)

        pltpu.emit_pipeline(
            body,
            grid=(n // _GW,),
            in_specs=[pl.BlockSpec((1, _GW), lambda i: (0, i)),
                      pl.BlockSpec((1, _GW), lambda i: (0, i))],
            out_specs=[pl.BlockSpec((_GW, 128), lambda i: (i, 0)),
                       pl.BlockSpec((_GW, 128), lambda i: (i, 0)),
                       pl.BlockSpec((_GW, 128), lambda i: (i, 0))],
            core_axis_name=("core", "subcore"),
            dimension_semantics=(pltpu.PARALLEL,),
        )(xi_hbm, pi_hbm, xg_hbm, lg_hbm, tg_hbm)

    return gather_kernel(xt8, lonp, latp, xidx, pidx)


def _epilogue_body(xg_ref, lg_ref, tg_ref, idx_ref, dist_ref,
                   xn_ref, xi_ref, lon_ref, lat_ref):
    rt, nh = idx_ref.shape
    b = xi_ref.shape[1]
    idx3 = idx_ref[...][:, :, None]
    lane = jax.lax.broadcasted_iota(jnp.int32, (rt, nh, _L), 2)
    sel_l = lane == idx3 % _L
    lon_ref[...] = jnp.sum(jnp.where(sel_l, lg_ref[...], 0.0), axis=2)
    lat_ref[...] = jnp.sum(jnp.where(sel_l, tg_ref[...], 0.0), axis=2)
    sel_x = (lane // b) == idx3 % 8
    xm = jnp.where(sel_x, xg_ref[...], 0.0)
    xn = _tree(jnp.add, [xm[:, :, g * b:(g + 1) * b] for g in range(8)])
    xn_ref[...] = xn
    w = 1.0 / (dist_ref[...] + _EPS)
    w = w / jnp.sum(w, axis=1, keepdims=True)
    xi_ref[...] = jnp.sum(xn * w[:, :, None], axis=1)


def _epilogue(xg, lg, tg, idx, dist_sel, b):
    t, nh = idx.shape
    rt = min(128, t)
    spec_g = pl.BlockSpec((rt, nh, 128), lambda i: (i, 0, 0))
    spec_s = pl.BlockSpec((rt, nh), lambda i: (i, 0))
    return pl.pallas_call(
        _epilogue_body,
        grid=(t // rt,),
        in_specs=[spec_g, spec_g, spec_g, spec_s, spec_s],
        out_specs=[pl.BlockSpec((rt, nh, b), lambda i: (i, 0, 0)),
                   pl.BlockSpec((rt, b), lambda i: (i, 0)), spec_s, spec_s],
        out_shape=(jax.ShapeDtypeStruct((t, nh, b), jnp.float32),
                   jax.ShapeDtypeStruct((t, b), jnp.float32),
                   jax.ShapeDtypeStruct((t, nh), jnp.float32),
                   jax.ShapeDtypeStruct((t, nh), jnp.float32)),
    )(xg.reshape(t, nh, 128), lg.reshape(t, nh, 128), tg.reshape(t, nh, 128),
      idx, dist_sel)


def kernel(x, d_lon, d_lat):
    b, s = x.shape
    t = d_lon.shape[0]
    dist_sel, idx, pidx, xidx = _select(d_lon, d_lat)
    xt8 = x.T.reshape(s // 8, 8 * b)
    lonp = d_lon.reshape(t * s // _L, _L)
    latp = d_lat.reshape(t * s // _L, _L)
    xg, lg, tg = _sc_gather(xt8, lonp, latp,
                            xidx.reshape(1, t * _NH), pidx.reshape(1, t * _NH))
    xn, xi_t, lon_sel, lat_sel = _epilogue(xg, lg, tg, idx, dist_sel, b)
    x_nearest = xn.transpose(2, 0, 1)
    x_inter = xi_t.T
    return (x_nearest, x_inter, dist_sel, lon_sel, lat_sel)


# speedup vs baseline: 2.1749x; 2.1749x over previous
import jax
import jax.numpy as jnp
from jax.experimental import pallas as pl
from jax.experimental.pallas import tpu as pltpu
from jax.experimental.pallas import tpu_sc as plsc

_NH = 16
_EPS = 1e-10
_L = 128
_NG = 4
_GW = 256


def _tree(op, xs):
    xs = list(xs)
    while len(xs) > 1:
        nxt = [op(xs[i], xs[i + 1]) for i in range(0, len(xs) - 1, 2)]
        if len(xs) % 2:
            nxt.append(xs[-1])
        xs = nxt
    return xs[0]


def _top2_tree(items):
    def comb(a, b):
        a1, pa1, alon1, alat1, a2, pa2, alon2, alat2 = a
        b1, pb1, blon1, blat1, b2, pb2, blon2, blat2 = b
        c1 = a1 <= b1
        w = lambda x, y: jnp.where(c1, x, y)
        m1, q1, mlon1, mlat1 = w(a1, b1), w(pa1, pb1), w(alon1, blon1), w(alat1, blat1)
        lo, plo, lolon, lolat = w(b1, a1), w(pb1, pa1), w(blon1, alon1), w(blat1, alat1)
        c2 = a2 <= b2
        w2 = lambda x, y: jnp.where(c2, x, y)
        m2a, q2a, lon2a, lat2a = w2(a2, b2), w2(pa2, pb2), w2(alon2, blon2), w2(alat2, blat2)
        c3 = lo <= m2a
        w3 = lambda x, y: jnp.where(c3, x, y)
        m2, q2, mlon2, mlat2 = w3(lo, m2a), w3(plo, q2a), w3(lolon, lon2a), w3(lolat, lat2a)
        return m1, q1, mlon1, mlat1, m2, q2, mlon2, mlat2
    xs = list(items)
    while len(xs) > 1:
        nxt = [comb(xs[i], xs[i + 1]) for i in range(0, len(xs) - 1, 2)]
        if len(xs) % 2:
            nxt.append(xs[-1])
        xs = nxt
    return xs[0]


def _select_body(lon_ref, lat_ref, dist_out, idx_out, xidx_out, lon_out,
                 lat_out, sq_ref):
    lon = lon_ref[...]
    lat = lat_ref[...]
    r, s = lon.shape
    nc = s // _L
    gc = nc // _NG
    sq = lon * lon + lat * lat
    lane_iota = jax.lax.broadcasted_iota(jnp.int32, (r, _L), 1)
    big_i = jnp.int32(2 ** 30)
    inf = jnp.float32(jnp.inf)
    zf = jnp.float32(0.0)

    chunks = [sq[:, c * _L:(c + 1) * _L] for c in range(nc)]
    lon_c = [lon[:, c * _L:(c + 1) * _L] for c in range(nc)]
    lat_c = [lat[:, c * _L:(c + 1) * _L] for c in range(nc)]

    def merge(state, cvs, cis, clons, clats):
        bv, bi, blon, blat = state
        vv = jnp.concatenate([bv] + cvs, axis=1)
        ii = jnp.concatenate([bi] + cis, axis=1)
        ll = jnp.concatenate([blon] + clons, axis=1)
        tt = jnp.concatenate([blat] + clats, axis=1)
        nbv, nbi, nbl, nbt = [], [], [], []
        for _ in range(_NH):
            m = jnp.min(vv, axis=1, keepdims=True)
            am = jnp.min(jnp.where(vv == m, ii, big_i), axis=1, keepdims=True)
            selm = ii == am
            nbv.append(m)
            nbi.append(am)
            nbl.append(jnp.sum(jnp.where(selm, ll, 0.0), axis=1, keepdims=True))
            nbt.append(jnp.sum(jnp.where(selm, tt, 0.0), axis=1, keepdims=True))
            vv = jnp.where(selm, jnp.inf, vv)
        return (jnp.concatenate(nbv, 1), jnp.concatenate(nbi, 1),
                jnp.concatenate(nbl, 1), jnp.concatenate(nbt, 1))

    tops = [_top2_tree([(chunks[g * gc + c], jnp.int32(g * gc + c),
                         lon_c[g * gc + c], lat_c[g * gc + c],
                         inf, big_i, zf, zf) for c in range(gc)])
            for g in range(_NG)]
    cvs = [t[0] for t in tops] + [t[4] for t in tops]
    cis = [t[1] * _L + lane_iota for t in tops] + \
          [t[5] * _L + lane_iota for t in tops]
    clons = [t[2] for t in tops] + [t[6] for t in tops]
    clats = [t[3] for t in tops] + [t[7] for t in tops]
    state = merge((jnp.full((r, _NH), jnp.inf, jnp.float32),
                   jnp.zeros((r, _NH), jnp.int32),
                   jnp.zeros((r, _NH), jnp.float32),
                   jnp.zeros((r, _NH), jnp.float32)), cvs, cis, clons, clats)

    masked = [jnp.where((tops[c // gc][1] == c) | (tops[c // gc][5] == c),
                        jnp.inf, chunks[c]) for c in range(nc)]
    sq_ref[...] = jnp.concatenate(masked, axis=1)

    def group_mins(ch):
        return [_tree(jnp.minimum, ch[g * gc:(g + 1) * gc])
                for g in range(_NG)]

    lmall = jnp.concatenate(group_mins(masked), axis=1)
    done = jnp.all(state[0][:, _NH - 1] < jnp.min(lmall, axis=1))

    def body(carry):
        _, lmall, bv, bi, blon, blat = carry
        gm = [lmall[:, g * _L:(g + 1) * _L] for g in range(_NG)]
        ch = [sq_ref[:, c * _L:(c + 1) * _L] for c in range(nc)]
        hit = [ch[c] == gm[c // gc] for c in range(nc)]
        pos = [_tree(jnp.minimum,
                     [jnp.where(hit[g * gc + c], g * gc + c, big_i)
                      for c in range(gc)]) for g in range(_NG)]
        sel = [pos[c // gc] == c for c in range(nc)]
        clon = [_tree(jnp.add,
                      [jnp.where(sel[g * gc + c], lon_c[g * gc + c], 0.0)
                       for c in range(gc)]) for g in range(_NG)]
        clat = [_tree(jnp.add,
                      [jnp.where(sel[g * gc + c], lat_c[g * gc + c], 0.0)
                       for c in range(gc)]) for g in range(_NG)]
        new_ch = [jnp.where(sel[c], jnp.inf, ch[c]) for c in range(nc)]
        sq_ref[...] = jnp.concatenate(new_ch, axis=1)
        bv, bi, blon, blat = merge((bv, bi, blon, blat), gm,
                                   [p * _L + lane_iota for p in pos],
                                   clon, clat)
        lmall = jnp.concatenate(group_mins(new_ch), axis=1)
        done = jnp.all(bv[:, _NH - 1] < jnp.min(lmall, axis=1))
        return done, lmall, bv, bi, blon, blat

    out = jax.lax.while_loop(lambda c: jnp.logical_not(c[0]), body,
                             (done, lmall) + state)
    _, _, bv, bi, blon, blat = out
    dist_out[...] = jnp.sqrt(bv + 1e-12)
    idx_out[...] = bi
    xidx_out[...] = bi // 8
    lon_out[...] = blon
    lat_out[...] = blat


def _select(d_lon, d_lat):
    t, s = d_lon.shape
    r = min(32, t)
    in_spec = pl.BlockSpec((r, s), lambda i: (i, 0))
    out_spec = pl.BlockSpec((r, _NH), lambda i: (i, 0))
    return pl.pallas_call(
        _select_body,
        grid=(t // r,),
        in_specs=[in_spec, in_spec],
        out_specs=[out_spec] * 5,
        out_shape=(jax.ShapeDtypeStruct((t, _NH), jnp.float32),
                   jax.ShapeDtypeStruct((t, _NH), jnp.int32),
                   jax.ShapeDtypeStruct((t, _NH), jnp.int32),
                   jax.ShapeDtypeStruct((t, _NH), jnp.float32),
                   jax.ShapeDtypeStruct((t, _NH), jnp.float32)),
        scratch_shapes=[pltpu.VMEM((r, s), jnp.float32)],
    )(d_lon, d_lat)


def _sc_gather(xt8, xidx):
    n = xidx.shape[1]
    mesh = plsc.VectorSubcoreMesh(core_axis_name="core",
                                  subcore_axis_name="subcore")

    @pl.kernel(out_type=jax.ShapeDtypeStruct((n, 128), jnp.float32),
               mesh=mesh)
    def gather_kernel(xt_hbm, xi_hbm, xg_hbm):
        def body(xi_vmem, xg_vmem):
            pltpu.sync_copy(xt_hbm.at[xi_vmem.at[0]], xg_vmem)

        pltpu.emit_pipeline(
            body,
            grid=(n // _GW,),
            in_specs=[pl.BlockSpec((1, _GW), lambda i: (0, i))],
            out_specs=[pl.BlockSpec((_GW, 128), lambda i: (i, 0))],
            core_axis_name=("core", "subcore"),
            dimension_semantics=(pltpu.PARALLEL,),
        )(xi_hbm, xg_hbm)

    return gather_kernel(xt8, xidx)


def _epilogue_body(xg_ref, idx_ref, dist_ref, xn_ref, xi_ref):
    rt, nh = idx_ref.shape
    b = xi_ref.shape[1]
    idx3 = idx_ref[...][:, :, None]
    lane = jax.lax.broadcasted_iota(jnp.int32, (rt, nh, _L), 2)
    sel_x = (lane // b) == idx3 % 8
    xm = jnp.where(sel_x, xg_ref[...], 0.0)
    xn = _tree(jnp.add, [xm[:, :, g * b:(g + 1) * b] for g in range(8)])
    xn_ref[...] = xn
    w = 1.0 / (dist_ref[...] + _EPS)
    w = w / jnp.sum(w, axis=1, keepdims=True)
    xi_ref[...] = jnp.sum(xn * w[:, :, None], axis=1)


def _epilogue(xg, idx, dist_sel, b):
    t, nh = idx.shape
    rt = min(128, t)
    spec_s = pl.BlockSpec((rt, nh), lambda i: (i, 0))
    return pl.pallas_call(
        _epilogue_body,
        grid=(t // rt,),
        in_specs=[pl.BlockSpec((rt, nh, 128), lambda i: (i, 0, 0)),
                  spec_s, spec_s],
        out_specs=[pl.BlockSpec((rt, nh, b), lambda i: (i, 0, 0)),
                   pl.BlockSpec((rt, b), lambda i: (i, 0))],
        out_shape=(jax.ShapeDtypeStruct((t, nh, b), jnp.float32),
                   jax.ShapeDtypeStruct((t, b), jnp.float32)),
    )(xg.reshape(t, nh, 128), idx, dist_sel)


def kernel(x, d_lon, d_lat):
    b, s = x.shape
    t = d_lon.shape[0]
    dist_sel, idx, xidx, lon_sel, lat_sel = _select(d_lon, d_lat)
    xt8 = x.T.reshape(s // 8, 8 * b)
    xg = _sc_gather(xt8, xidx.reshape(1, t * _NH))
    xn, xi_t, = _epilogue(xg, idx, dist_sel, b)
    x_nearest = xn.transpose(2, 0, 1)
    x_inter = xi_t.T
    return (x_nearest, x_inter, dist_sel, lon_sel, lat_sel)


# speedup vs baseline: 2.9573x; 1.3597x over previous
import jax
import jax.numpy as jnp
from jax.experimental import pallas as pl
from jax.experimental.pallas import tpu as pltpu
from jax.experimental.pallas import tpu_sc as plsc

_NH = 16
_EPS = 1e-10
_L = 128
_NG = 4
_GW = 256


def _tree(op, xs):
    xs = list(xs)
    while len(xs) > 1:
        nxt = [op(xs[i], xs[i + 1]) for i in range(0, len(xs) - 1, 2)]
        if len(xs) % 2:
            nxt.append(xs[-1])
        xs = nxt
    return xs[0]


def _top2_tree(items):
    def comb(a, b):
        a1, pa1, alon1, alat1, a2, pa2, alon2, alat2 = a
        b1, pb1, blon1, blat1, b2, pb2, blon2, blat2 = b
        c1 = a1 <= b1
        w = lambda x, y: jnp.where(c1, x, y)
        m1, q1, mlon1, mlat1 = w(a1, b1), w(pa1, pb1), w(alon1, blon1), w(alat1, blat1)
        lo, plo, lolon, lolat = w(b1, a1), w(pb1, pa1), w(blon1, alon1), w(blat1, alat1)
        c2 = a2 <= b2
        w2 = lambda x, y: jnp.where(c2, x, y)
        m2a, q2a, lon2a, lat2a = w2(a2, b2), w2(pa2, pb2), w2(alon2, blon2), w2(alat2, blat2)
        c3 = lo <= m2a
        w3 = lambda x, y: jnp.where(c3, x, y)
        m2, q2, mlon2, mlat2 = w3(lo, m2a), w3(plo, q2a), w3(lolon, lon2a), w3(lolat, lat2a)
        return m1, q1, mlon1, mlat1, m2, q2, mlon2, mlat2
    xs = list(items)
    while len(xs) > 1:
        nxt = [comb(xs[i], xs[i + 1]) for i in range(0, len(xs) - 1, 2)]
        if len(xs) % 2:
            nxt.append(xs[-1])
        xs = nxt
    return xs[0]


def _select_body(lon_ref, lat_ref, dist_out, idx_out, xidx_out, lon_out,
                 lat_out, sq_ref):
    lon = lon_ref[...]
    lat = lat_ref[...]
    r, s = lon.shape
    nc = s // _L
    gc = nc // _NG
    sq = lon * lon + lat * lat
    lane_iota = jax.lax.broadcasted_iota(jnp.int32, (r, _L), 1)
    big_i = jnp.int32(2 ** 30)
    inf = jnp.float32(jnp.inf)
    zf = jnp.float32(0.0)

    chunks = [sq[:, c * _L:(c + 1) * _L] for c in range(nc)]
    lon_c = [lon[:, c * _L:(c + 1) * _L] for c in range(nc)]
    lat_c = [lat[:, c * _L:(c + 1) * _L] for c in range(nc)]

    def merge(state, cvs, cis, clons, clats):
        bv, bi, blon, blat = state
        rg = min(16, r)
        outs = []
        for r0 in range(0, r, rg):
            sl = slice(r0, r0 + rg)
            vv = jnp.concatenate([bv[sl]] + [c[sl] for c in cvs], axis=1)
            ii = jnp.concatenate([bi[sl]] + [c[sl] for c in cis], axis=1)
            ll = jnp.concatenate([blon[sl]] + [c[sl] for c in clons], axis=1)
            tt = jnp.concatenate([blat[sl]] + [c[sl] for c in clats], axis=1)
            nbv, nbi, nbl, nbt = [], [], [], []
            for _ in range(_NH):
                m = jnp.min(vv, axis=1, keepdims=True)
                am = jnp.min(jnp.where(vv == m, ii, big_i), axis=1,
                             keepdims=True)
                selm = ii == am
                nbv.append(m)
                nbi.append(am)
                nbl.append(jnp.sum(jnp.where(selm, ll, 0.0), axis=1,
                                   keepdims=True))
                nbt.append(jnp.sum(jnp.where(selm, tt, 0.0), axis=1,
                                   keepdims=True))
                vv = jnp.where(selm, jnp.inf, vv)
            outs.append((jnp.concatenate(nbv, 1), jnp.concatenate(nbi, 1),
                         jnp.concatenate(nbl, 1), jnp.concatenate(nbt, 1)))
        return tuple(jnp.concatenate([o[i] for o in outs], axis=0)
                     for i in range(4))

    tops = [_top2_tree([(chunks[g * gc + c], jnp.int32(g * gc + c),
                         lon_c[g * gc + c], lat_c[g * gc + c],
                         inf, big_i, zf, zf) for c in range(gc)])
            for g in range(_NG)]
    cvs = [t[0] for t in tops] + [t[4] for t in tops]
    cis = [t[1] * _L + lane_iota for t in tops] + \
          [t[5] * _L + lane_iota for t in tops]
    clons = [t[2] for t in tops] + [t[6] for t in tops]
    clats = [t[3] for t in tops] + [t[7] for t in tops]
    state = merge((jnp.full((r, _NH), jnp.inf, jnp.float32),
                   jnp.zeros((r, _NH), jnp.int32),
                   jnp.zeros((r, _NH), jnp.float32),
                   jnp.zeros((r, _NH), jnp.float32)), cvs, cis, clons, clats)

    masked = [jnp.where((tops[c // gc][1] == c) | (tops[c // gc][5] == c),
                        jnp.inf, chunks[c]) for c in range(nc)]
    sq_ref[...] = jnp.concatenate(masked, axis=1)

    def group_mins(ch):
        return [_tree(jnp.minimum, ch[g * gc:(g + 1) * gc])
                for g in range(_NG)]

    lmall = jnp.concatenate(group_mins(masked), axis=1)
    done = jnp.all(state[0][:, _NH - 1] < jnp.min(lmall, axis=1))

    def body(carry):
        _, lmall, bv, bi, blon, blat = carry
        gm = [lmall[:, g * _L:(g + 1) * _L] for g in range(_NG)]
        ch = [sq_ref[:, c * _L:(c + 1) * _L] for c in range(nc)]
        hit = [ch[c] == gm[c // gc] for c in range(nc)]
        pos = [_tree(jnp.minimum,
                     [jnp.where(hit[g * gc + c], g * gc + c, big_i)
                      for c in range(gc)]) for g in range(_NG)]
        sel = [pos[c // gc] == c for c in range(nc)]
        clon = [_tree(jnp.add,
                      [jnp.where(sel[g * gc + c], lon_c[g * gc + c], 0.0)
                       for c in range(gc)]) for g in range(_NG)]
        clat = [_tree(jnp.add,
                      [jnp.where(sel[g * gc + c], lat_c[g * gc + c], 0.0)
                       for c in range(gc)]) for g in range(_NG)]
        new_ch = [jnp.where(sel[c], jnp.inf, ch[c]) for c in range(nc)]
        sq_ref[...] = jnp.concatenate(new_ch, axis=1)
        bv, bi, blon, blat = merge((bv, bi, blon, blat), gm,
                                   [p * _L + lane_iota for p in pos],
                                   clon, clat)
        lmall = jnp.concatenate(group_mins(new_ch), axis=1)
        done = jnp.all(bv[:, _NH - 1] < jnp.min(lmall, axis=1))
        return done, lmall, bv, bi, blon, blat

    out = jax.lax.while_loop(lambda c: jnp.logical_not(c[0]), body,
                             (done, lmall) + state)
    _, _, bv, bi, blon, blat = out
    dist_out[...] = jnp.sqrt(bv + 1e-12)
    idx_out[...] = bi
    xidx_out[...] = bi // 8
    lon_out[...] = blon
    lat_out[...] = blat


def _select(d_lon, d_lat):
    t, s = d_lon.shape
    r = min(64, t)
    in_spec = pl.BlockSpec((r, s), lambda i: (i, 0))
    out_spec = pl.BlockSpec((r, _NH), lambda i: (i, 0))
    return pl.pallas_call(
        _select_body,
        grid=(t // r,),
        in_specs=[in_spec, in_spec],
        out_specs=[out_spec] * 5,
        out_shape=(jax.ShapeDtypeStruct((t, _NH), jnp.float32),
                   jax.ShapeDtypeStruct((t, _NH), jnp.int32),
                   jax.ShapeDtypeStruct((t, _NH), jnp.int32),
                   jax.ShapeDtypeStruct((t, _NH), jnp.float32),
                   jax.ShapeDtypeStruct((t, _NH), jnp.float32)),
        scratch_shapes=[pltpu.VMEM((r, s), jnp.float32)],
    )(d_lon, d_lat)


def _sc_gather(xt8, xidx):
    n = xidx.shape[1]
    mesh = plsc.VectorSubcoreMesh(core_axis_name="core",
                                  subcore_axis_name="subcore")

    @pl.kernel(out_type=jax.ShapeDtypeStruct((n, 128), jnp.float32),
               mesh=mesh)
    def gather_kernel(xt_hbm, xi_hbm, xg_hbm):
        def body(xi_vmem, xg_vmem):
            pltpu.sync_copy(xt_hbm.at[xi_vmem.at[0]], xg_vmem)

        pltpu.emit_pipeline(
            body,
            grid=(n // _GW,),
            in_specs=[pl.BlockSpec((1, _GW), lambda i: (0, i))],
            out_specs=[pl.BlockSpec((_GW, 128), lambda i: (i, 0))],
            core_axis_name=("core", "subcore"),
            dimension_semantics=(pltpu.PARALLEL,),
        )(xi_hbm, xg_hbm)

    return gather_kernel(xt8, xidx)


def _epilogue_body(xg_ref, idx_ref, dist_ref, xn_ref, xi_ref):
    rt, nh = idx_ref.shape
    b = xi_ref.shape[1]
    idx3 = idx_ref[...][:, :, None]
    lane = jax.lax.broadcasted_iota(jnp.int32, (rt, nh, _L), 2)
    sel_x = (lane // b) == idx3 % 8
    xm = jnp.where(sel_x, xg_ref[...], 0.0)
    xn = _tree(jnp.add, [xm[:, :, g * b:(g + 1) * b] for g in range(8)])
    xn_ref[...] = xn
    w = 1.0 / (dist_ref[...] + _EPS)
    w = w / jnp.sum(w, axis=1, keepdims=True)
    xi_ref[...] = jnp.sum(xn * w[:, :, None], axis=1)


def _epilogue(xg, idx, dist_sel, b):
    t, nh = idx.shape
    rt = min(128, t)
    spec_s = pl.BlockSpec((rt, nh), lambda i: (i, 0))
    return pl.pallas_call(
        _epilogue_body,
        grid=(t // rt,),
        in_specs=[pl.BlockSpec((rt, nh, 128), lambda i: (i, 0, 0)),
                  spec_s, spec_s],
        out_specs=[pl.BlockSpec((rt, nh, b), lambda i: (i, 0, 0)),
                   pl.BlockSpec((rt, b), lambda i: (i, 0))],
        out_shape=(jax.ShapeDtypeStruct((t, nh, b), jnp.float32),
                   jax.ShapeDtypeStruct((t, b), jnp.float32)),
    )(xg.reshape(t, nh, 128), idx, dist_sel)


def kernel(x, d_lon, d_lat):
    b, s = x.shape
    t = d_lon.shape[0]
    dist_sel, idx, xidx, lon_sel, lat_sel = _select(d_lon, d_lat)
    xt8 = x.T.reshape(s // 8, 8 * b)
    xg = _sc_gather(xt8, xidx.reshape(1, t * _NH))
    xn, xi_t, = _epilogue(xg, idx, dist_sel, b)
    x_nearest = xn.transpose(2, 0, 1)
    x_inter = xi_t.T
    return (x_nearest, x_inter, dist_sel, lon_sel, lat_sel)


# speedup vs baseline: 3.0193x; 1.0210x over previous
import jax
import jax.numpy as jnp
from jax.experimental import pallas as pl
from jax.experimental.pallas import tpu as pltpu
from jax.experimental.pallas import tpu_sc as plsc

_NH = 16
_EPS = 1e-10
_L = 128
_NG = 4
_GW = 256


def _tree(op, xs):
    xs = list(xs)
    while len(xs) > 1:
        nxt = [op(xs[i], xs[i + 1]) for i in range(0, len(xs) - 1, 2)]
        if len(xs) % 2:
            nxt.append(xs[-1])
        xs = nxt
    return xs[0]


def _top2_tree(items):
    def comb(a, b):
        a1, pa1, alon1, alat1, a2, pa2, alon2, alat2 = a
        b1, pb1, blon1, blat1, b2, pb2, blon2, blat2 = b
        c1 = a1 <= b1
        w = lambda x, y: jnp.where(c1, x, y)
        m1, q1, mlon1, mlat1 = w(a1, b1), w(pa1, pb1), w(alon1, blon1), w(alat1, blat1)
        lo, plo, lolon, lolat = w(b1, a1), w(pb1, pa1), w(blon1, alon1), w(blat1, alat1)
        c2 = a2 <= b2
        w2 = lambda x, y: jnp.where(c2, x, y)
        m2a, q2a, lon2a, lat2a = w2(a2, b2), w2(pa2, pb2), w2(alon2, blon2), w2(alat2, blat2)
        c3 = lo <= m2a
        w3 = lambda x, y: jnp.where(c3, x, y)
        m2, q2, mlon2, mlat2 = w3(lo, m2a), w3(plo, q2a), w3(lolon, lon2a), w3(lolat, lat2a)
        return m1, q1, mlon1, mlat1, m2, q2, mlon2, mlat2
    xs = list(items)
    while len(xs) > 1:
        nxt = [comb(xs[i], xs[i + 1]) for i in range(0, len(xs) - 1, 2)]
        if len(xs) % 2:
            nxt.append(xs[-1])
        xs = nxt
    return xs[0]


def _select_body(lon_ref, lat_ref, dist_out, idx_out, xidx_out, lon_out,
                 lat_out, sq_ref):
    lon = lon_ref[...]
    lat = lat_ref[...]
    r, s = lon.shape
    nc = s // _L
    gc = nc // _NG
    sq = lon * lon + lat * lat
    lane_iota = jax.lax.broadcasted_iota(jnp.int32, (r, _L), 1)
    big_i = jnp.int32(2 ** 30)
    inf = jnp.float32(jnp.inf)
    zf = jnp.float32(0.0)

    chunks = [sq[:, c * _L:(c + 1) * _L] for c in range(nc)]
    lon_c = [lon[:, c * _L:(c + 1) * _L] for c in range(nc)]
    lat_c = [lat[:, c * _L:(c + 1) * _L] for c in range(nc)]

    def merge(state, cvs, cis, clons, clats):
        bv, bi, blon, blat = state
        rg = min(8, r)
        outs = []
        for r0 in range(0, r, rg):
            sl = slice(r0, r0 + rg)
            vv = jnp.concatenate([bv[sl]] + [c[sl] for c in cvs], axis=1)
            ii = jnp.concatenate([bi[sl]] + [c[sl] for c in cis], axis=1)
            ll = jnp.concatenate([blon[sl]] + [c[sl] for c in clons], axis=1)
            tt = jnp.concatenate([blat[sl]] + [c[sl] for c in clats], axis=1)
            nbv, nbi, nbl, nbt = [], [], [], []
            for _ in range(_NH):
                m = jnp.min(vv, axis=1, keepdims=True)
                am = jnp.min(jnp.where(vv == m, ii, big_i), axis=1,
                             keepdims=True)
                selm = ii == am
                nbv.append(m)
                nbi.append(am)
                nbl.append(jnp.sum(jnp.where(selm, ll, 0.0), axis=1,
                                   keepdims=True))
                nbt.append(jnp.sum(jnp.where(selm, tt, 0.0), axis=1,
                                   keepdims=True))
                vv = jnp.where(selm, jnp.inf, vv)
            outs.append((jnp.concatenate(nbv, 1), jnp.concatenate(nbi, 1),
                         jnp.concatenate(nbl, 1), jnp.concatenate(nbt, 1)))
        return tuple(jnp.concatenate([o[i] for o in outs], axis=0)
                     for i in range(4))

    tops = [_top2_tree([(chunks[g * gc + c], jnp.int32(g * gc + c),
                         lon_c[g * gc + c], lat_c[g * gc + c],
                         inf, big_i, zf, zf) for c in range(gc)])
            for g in range(_NG)]
    cvs = [t[0] for t in tops] + [t[4] for t in tops]
    cis = [t[1] * _L + lane_iota for t in tops] + \
          [t[5] * _L + lane_iota for t in tops]
    clons = [t[2] for t in tops] + [t[6] for t in tops]
    clats = [t[3] for t in tops] + [t[7] for t in tops]
    state = merge((jnp.full((r, _NH), jnp.inf, jnp.float32),
                   jnp.zeros((r, _NH), jnp.int32),
                   jnp.zeros((r, _NH), jnp.float32),
                   jnp.zeros((r, _NH), jnp.float32)), cvs, cis, clons, clats)

    masked = [jnp.where((tops[c // gc][1] == c) | (tops[c // gc][5] == c),
                        jnp.inf, chunks[c]) for c in range(nc)]
    sq_ref[...] = jnp.concatenate(masked, axis=1)

    def group_mins(ch):
        return [_tree(jnp.minimum, ch[g * gc:(g + 1) * gc])
                for g in range(_NG)]

    lmall = jnp.concatenate(group_mins(masked), axis=1)
    done = jnp.all(state[0][:, _NH - 1] < jnp.min(lmall, axis=1))

    def body(carry):
        _, lmall, bv, bi, blon, blat = carry
        gm = [lmall[:, g * _L:(g + 1) * _L] for g in range(_NG)]
        ch = [sq_ref[:, c * _L:(c + 1) * _L] for c in range(nc)]
        hit = [ch[c] == gm[c // gc] for c in range(nc)]
        pos = [_tree(jnp.minimum,
                     [jnp.where(hit[g * gc + c], g * gc + c, big_i)
                      for c in range(gc)]) for g in range(_NG)]
        sel = [pos[c // gc] == c for c in range(nc)]
        clon = [_tree(jnp.add,
                      [jnp.where(sel[g * gc + c], lon_c[g * gc + c], 0.0)
                       for c in range(gc)]) for g in range(_NG)]
        clat = [_tree(jnp.add,
                      [jnp.where(sel[g * gc + c], lat_c[g * gc + c], 0.0)
                       for c in range(gc)]) for g in range(_NG)]
        new_ch = [jnp.where(sel[c], jnp.inf, ch[c]) for c in range(nc)]
        sq_ref[...] = jnp.concatenate(new_ch, axis=1)
        bv, bi, blon, blat = merge((bv, bi, blon, blat), gm,
                                   [p * _L + lane_iota for p in pos],
                                   clon, clat)
        lmall = jnp.concatenate(group_mins(new_ch), axis=1)
        done = jnp.all(bv[:, _NH - 1] < jnp.min(lmall, axis=1))
        return done, lmall, bv, bi, blon, blat

    out = jax.lax.while_loop(lambda c: jnp.logical_not(c[0]), body,
                             (done, lmall) + state)
    _, _, bv, bi, blon, blat = out
    dist_out[...] = jnp.sqrt(bv + 1e-12)
    idx_out[...] = bi
    xidx_out[...] = bi // 8
    lon_out[...] = blon
    lat_out[...] = blat


def _select(d_lon, d_lat):
    t, s = d_lon.shape
    r = min(64, t)
    in_spec = pl.BlockSpec((r, s), lambda i: (i, 0))
    out_spec = pl.BlockSpec((r, _NH), lambda i: (i, 0))
    return pl.pallas_call(
        _select_body,
        grid=(t // r,),
        in_specs=[in_spec, in_spec],
        out_specs=[out_spec] * 5,
        out_shape=(jax.ShapeDtypeStruct((t, _NH), jnp.float32),
                   jax.ShapeDtypeStruct((t, _NH), jnp.int32),
                   jax.ShapeDtypeStruct((t, _NH), jnp.int32),
                   jax.ShapeDtypeStruct((t, _NH), jnp.float32),
                   jax.ShapeDtypeStruct((t, _NH), jnp.float32)),
        scratch_shapes=[pltpu.VMEM((r, s), jnp.float32)],
    )(d_lon, d_lat)


def _sc_gather(xt8, xidx):
    n = xidx.shape[1]
    mesh = plsc.VectorSubcoreMesh(core_axis_name="core",
                                  subcore_axis_name="subcore")

    @pl.kernel(out_type=jax.ShapeDtypeStruct((n, 128), jnp.float32),
               mesh=mesh)
    def gather_kernel(xt_hbm, xi_hbm, xg_hbm):
        def body(xi_vmem, xg_vmem):
            pltpu.sync_copy(xt_hbm.at[xi_vmem.at[0]], xg_vmem)

        pltpu.emit_pipeline(
            body,
            grid=(n // _GW,),
            in_specs=[pl.BlockSpec((1, _GW), lambda i: (0, i))],
            out_specs=[pl.BlockSpec((_GW, 128), lambda i: (i, 0))],
            core_axis_name=("core", "subcore"),
            dimension_semantics=(pltpu.PARALLEL,),
        )(xi_hbm, xg_hbm)

    return gather_kernel(xt8, xidx)


def _epilogue_body(xg_ref, idx_ref, dist_ref, xn_ref, xi_ref):
    rt, nh = idx_ref.shape
    b = xi_ref.shape[1]
    idx3 = idx_ref[...][:, :, None]
    lane = jax.lax.broadcasted_iota(jnp.int32, (rt, nh, _L), 2)
    sel_x = (lane // b) == idx3 % 8
    xm = jnp.where(sel_x, xg_ref[...], 0.0)
    xn = _tree(jnp.add, [xm[:, :, g * b:(g + 1) * b] for g in range(8)])
    xn_ref[...] = xn
    w = 1.0 / (dist_ref[...] + _EPS)
    w = w / jnp.sum(w, axis=1, keepdims=True)
    xi_ref[...] = jnp.sum(xn * w[:, :, None], axis=1)


def _epilogue(xg, idx, dist_sel, b):
    t, nh = idx.shape
    rt = min(128, t)
    spec_s = pl.BlockSpec((rt, nh), lambda i: (i, 0))
    return pl.pallas_call(
        _epilogue_body,
        grid=(t // rt,),
        in_specs=[pl.BlockSpec((rt, nh, 128), lambda i: (i, 0, 0)),
                  spec_s, spec_s],
        out_specs=[pl.BlockSpec((rt, nh, b), lambda i: (i, 0, 0)),
                   pl.BlockSpec((rt, b), lambda i: (i, 0))],
        out_shape=(jax.ShapeDtypeStruct((t, nh, b), jnp.float32),
                   jax.ShapeDtypeStruct((t, b), jnp.float32)),
    )(xg.reshape(t, nh, 128), idx, dist_sel)


def kernel(x, d_lon, d_lat):
    b, s = x.shape
    t = d_lon.shape[0]
    dist_sel, idx, xidx, lon_sel, lat_sel = _select(d_lon, d_lat)
    xt8 = x.T.reshape(s // 8, 8 * b)
    xg = _sc_gather(xt8, xidx.reshape(1, t * _NH))
    xn, xi_t, = _epilogue(xg, idx, dist_sel, b)
    x_nearest = xn.transpose(2, 0, 1)
    x_inter = xi_t.T
    return (x_nearest, x_inter, dist_sel, lon_sel, lat_sel)
